# baseline (speedup 1.0000x reference)
import functools

import jax
import jax.numpy as jnp
from jax import lax
from jax.experimental import pallas as pl
from jax.experimental.pallas import tpu as pltpu
from jax.experimental.pallas import tpu_sc as plsc

_D = 32
_B = 16384
_L = 16

_info = plsc.get_sparse_core_info()
_NC, _NS = _info.num_cores, _info.num_subcores
_NW = _NC * _NS
_BPW = _B // _NW
_CHUNK = 128
_NCHUNK = _BPW // _CHUNK


def _cf_body(uid_hbm, pid_hbm, ut_hbm, pt_hbm, out_hbm,
             uid_v, pid_v, urows, prows, outc, sem_u, sem_p):
    wid = lax.axis_index("s") * _NC + lax.axis_index("c")
    pltpu.sync_copy(uid_hbm.at[pl.ds(wid * _NCHUNK, _NCHUNK)], uid_v)
    pltpu.sync_copy(pid_hbm.at[pl.ds(wid * _NCHUNK, _NCHUNK)], pid_v)
    copies = []
    for j in range(_NCHUNK):
        copies.append(pltpu.async_copy(
            ut_hbm.at[uid_v.at[j]], urows.at[pl.ds(j * _CHUNK, _CHUNK)], sem_u))
        copies.append(pltpu.async_copy(
            pt_hbm.at[pid_v.at[j]], prows.at[pl.ds(j * _CHUNK, _CHUNK)], sem_p))
    for c in copies:
        c.wait()

    lanes = lax.iota(jnp.int32, _L)

    def group(g, carry):
        base = g * _L
        acc = jnp.zeros((_L,), jnp.float32)
        for i in range(_L):
            b = base + i
            wu = plsc.bitcast(urows[b, pl.ds(0, 2 * _L)], jnp.int32)
            wp = plsc.bitcast(prows[b, pl.ds(0, 2 * _L)], jnp.int32)
            lou = plsc.bitcast(wu << 16, jnp.float32)
            lop = plsc.bitcast(wp << 16, jnp.float32)
            hiu = plsc.bitcast(wu & jnp.int32(-65536), jnp.float32)
            hip = plsc.bitcast(wp & jnp.int32(-65536), jnp.float32)
            tot = jnp.sum(lou * lop + hiu * hip)
            acc = jnp.where(lanes == i, tot, acc)
        outc[pl.ds(base, _L)] = 1.0 / (1.0 + jnp.exp(-acc))
        return carry

    lax.fori_loop(0, _BPW // _L, group, 0)
    pltpu.sync_copy(outc, out_hbm.at[pl.ds(wid * _BPW, _BPW)])


@jax.jit
def kernel(user_ids, post_ids, user_table, post_table):
    uid = user_ids.astype(jnp.int32).reshape(_B // _CHUNK, _CHUNK)
    pid = post_ids.astype(jnp.int32).reshape(_B // _CHUNK, _CHUNK)
    mesh = plsc.VectorSubcoreMesh(core_axis_name="c", subcore_axis_name="s")
    f = pl.kernel(
        _cf_body,
        out_type=jax.ShapeDtypeStruct((_B,), jnp.float32),
        mesh=mesh,
        compiler_params=pltpu.CompilerParams(
            needs_layout_passes=False, use_tc_tiling_on_sc=False),
        scratch_types=[
            pltpu.VMEM((_NCHUNK, _CHUNK), jnp.int32),
            pltpu.VMEM((_NCHUNK, _CHUNK), jnp.int32),
            pltpu.VMEM((_BPW, _D), jnp.bfloat16),
            pltpu.VMEM((_BPW, _D), jnp.bfloat16),
            pltpu.VMEM((_BPW,), jnp.float32),
            pltpu.SemaphoreType.DMA,
            pltpu.SemaphoreType.DMA,
        ],
    )
    return f(uid, pid,
             user_table.astype(jnp.bfloat16), post_table.astype(jnp.bfloat16))


# speedup vs baseline: 1.1740x; 1.1740x over previous
import functools

import jax
import jax.numpy as jnp
from jax import lax
from jax.experimental import pallas as pl
from jax.experimental.pallas import tpu as pltpu
from jax.experimental.pallas import tpu_sc as plsc

_D = 32
_B = 16384
_L = 16

_info = plsc.get_sparse_core_info()
_NC, _NS = _info.num_cores, _info.num_subcores
_NW = _NC * _NS
_BPW = _B // _NW
_CHUNK = 128
_NCHUNK = _BPW // _CHUNK


def _cf_body(uid_hbm, pid_hbm, ut_hbm, pt_hbm, out_hbm,
             uid_v, pid_v, urows, prows, outc, sem_u, sem_p):
    wid = lax.axis_index("s") * _NC + lax.axis_index("c")
    pltpu.sync_copy(uid_hbm.at[pl.ds(wid * _NCHUNK, _NCHUNK)], uid_v)
    pltpu.sync_copy(pid_hbm.at[pl.ds(wid * _NCHUNK, _NCHUNK)], pid_v)
    copies = []
    for j in range(_NCHUNK):
        copies.append(pltpu.async_copy(
            ut_hbm.at[uid_v.at[j]], urows.at[pl.ds(j * _CHUNK, _CHUNK)], sem_u))
        copies.append(pltpu.async_copy(
            pt_hbm.at[pid_v.at[j]], prows.at[pl.ds(j * _CHUNK, _CHUNK)], sem_p))
    for c in copies:
        c.wait()

    lanes = lax.iota(jnp.int32, _L)

    def group(g, carry):
        base = g * _L
        acc = jnp.zeros((_L,), jnp.float32)
        for i in range(_L):
            b = base + i
            u0 = urows[b, pl.ds(0, _L)]
            u1 = urows[b, pl.ds(_L, _L)]
            p0 = prows[b, pl.ds(0, _L)]
            p1 = prows[b, pl.ds(_L, _L)]
            tot = jnp.sum(u0 * p0 + u1 * p1)
            acc = jnp.where(lanes == i, tot, acc)
        outc[pl.ds(base, _L)] = 1.0 / (1.0 + jnp.exp(-acc))
        return carry

    lax.fori_loop(0, _BPW // _L, group, 0)
    pltpu.sync_copy(outc, out_hbm.at[pl.ds(wid * _BPW, _BPW)])


@jax.jit
def kernel(user_ids, post_ids, user_table, post_table):
    uid = user_ids.astype(jnp.int32).reshape(_B // _CHUNK, _CHUNK)
    pid = post_ids.astype(jnp.int32).reshape(_B // _CHUNK, _CHUNK)
    mesh = plsc.VectorSubcoreMesh(core_axis_name="c", subcore_axis_name="s")
    f = pl.kernel(
        _cf_body,
        out_type=jax.ShapeDtypeStruct((_B,), jnp.float32),
        mesh=mesh,
        compiler_params=pltpu.CompilerParams(
            needs_layout_passes=False, use_tc_tiling_on_sc=False),
        scratch_types=[
            pltpu.VMEM((_NCHUNK, _CHUNK), jnp.int32),
            pltpu.VMEM((_NCHUNK, _CHUNK), jnp.int32),
            pltpu.VMEM((_BPW, _D), jnp.float32),
            pltpu.VMEM((_BPW, _D), jnp.float32),
            pltpu.VMEM((_BPW,), jnp.float32),
            pltpu.SemaphoreType.DMA,
            pltpu.SemaphoreType.DMA,
        ],
    )
    return f(uid, pid, user_table, post_table)
